# speedup vs baseline: 1.0434x; 1.0434x over previous
import functools

import jax
import jax.numpy as jnp
from jax import lax
from jax.experimental import pallas as pl
from jax.experimental.pallas import tpu as pltpu
from jax.experimental.pallas import tpu_sc as plsc

DIM = 128
LANES = 16
CHUNK = 256
SUB = 128


def _emb_body(n_per_w, n_chunks, seq_len, num_cores,
              idx_hbm, tok_hbm, pos_hbm, out_hbm,
              idx_v, rows_v, pos_v, sem):
  cid = lax.axis_index("c")
  sid = lax.axis_index("s")
  wid = sid * num_cores + cid
  base = wid * n_per_w

  def chunk_body(g, carry):
    off = base + g * CHUNK
    t_off = lax.rem(off, seq_len)
    pltpu.sync_copy(idx_hbm.at[pl.ds(off, CHUNK)], idx_v)
    for j in range(CHUNK // SUB):
      pltpu.async_copy(
          tok_hbm.at[idx_v.at[pl.ds(j * SUB, SUB)]],
          rows_v.at[pl.ds(j * SUB, SUB)],
          sem,
      )
    pltpu.sync_copy(pos_hbm.at[pl.ds(t_off, CHUNK)], pos_v)
    for j in range(CHUNK // SUB):
      pltpu.make_async_copy(
          tok_hbm.at[idx_v.at[pl.ds(j * SUB, SUB)]],
          rows_v.at[pl.ds(j * SUB, SUB)],
          sem,
      ).wait()

    def add_row(i, c):
      for j in range(DIM // LANES):
        s = pl.ds(j * LANES, LANES)
        rows_v[i, s] = rows_v[i, s] + pos_v[i, s]
      return c

    lax.fori_loop(0, CHUNK, add_row, 0)
    pltpu.sync_copy(rows_v, out_hbm.at[pl.ds(off, CHUNK)])
    return carry

  lax.fori_loop(0, n_chunks, chunk_body, 0)


def kernel(idx, token_emb, pos_emb):
  B, T = idx.shape
  N = B * T
  info = plsc.get_sparse_core_info()
  num_workers = info.num_cores * info.num_subcores
  n_per_w = N // num_workers
  n_chunks = n_per_w // CHUNK

  idx_flat = idx.reshape(N).astype(jnp.int32)

  mesh = plsc.VectorSubcoreMesh(core_axis_name="c", subcore_axis_name="s")
  run = functools.partial(
      pl.kernel,
      mesh=mesh,
      out_type=jax.ShapeDtypeStruct((N, DIM), jnp.float32),
      scratch_types=[
          pltpu.VMEM((CHUNK,), jnp.int32),
          pltpu.VMEM((CHUNK, DIM), jnp.float32),
          pltpu.VMEM((CHUNK, DIM), jnp.float32),
          pltpu.SemaphoreType.DMA,
      ],
  )(functools.partial(_emb_body, n_per_w, n_chunks, T, info.num_cores))

  out = run(idx_flat, token_emb, pos_emb)
  return out.reshape(B, T, DIM)


# speedup vs baseline: 1.3855x; 1.3278x over previous
import functools

import jax
import jax.numpy as jnp
from jax import lax
from jax.experimental import pallas as pl
from jax.experimental.pallas import tpu as pltpu
from jax.experimental.pallas import tpu_sc as plsc

DIM = 128
LANES = 16
CHUNK = 256
SUB = 128


def _emb_body(t_per_w, seq_len, batch, num_cores,
              idx_hbm, tok_hbm, pos_hbm, out_hbm,
              idx0, idx1, rows0, rows1, pos_v, sem_g, sem_o):
  cid = lax.axis_index("c")
  sid = lax.axis_index("s")
  wid = sid * num_cores + cid
  t0 = wid * t_per_w

  idx_bufs = (idx0, idx1)
  row_bufs = (rows0, rows1)

  pltpu.sync_copy(pos_hbm.at[pl.ds(t0, CHUNK)], pos_v)

  def load_and_gather(k):
    b = k
    off = b * seq_len + t0
    idx_v = idx_bufs[k % 2]
    rows_v = row_bufs[k % 2]
    pltpu.sync_copy(idx_hbm.at[pl.ds(off, CHUNK)], idx_v)
    for j in range(CHUNK // SUB):
      pltpu.async_copy(
          tok_hbm.at[idx_v.at[pl.ds(j * SUB, SUB)]],
          rows_v.at[pl.ds(j * SUB, SUB)],
          sem_g,
      )

  def wait_gather(k):
    idx_v = idx_bufs[k % 2]
    rows_v = row_bufs[k % 2]
    for j in range(CHUNK // SUB):
      pltpu.make_async_copy(
          tok_hbm.at[idx_v.at[pl.ds(j * SUB, SUB)]],
          rows_v.at[pl.ds(j * SUB, SUB)],
          sem_g,
      ).wait()

  def write_out(k):
    off = k * seq_len + t0
    pltpu.async_copy(row_bufs[k % 2], out_hbm.at[pl.ds(off, CHUNK)], sem_o)

  def wait_out(k):
    off = k * seq_len + t0
    pltpu.make_async_copy(
        row_bufs[k % 2], out_hbm.at[pl.ds(off, CHUNK)], sem_o
    ).wait()

  def add_pos(k):
    rows_v = row_bufs[k % 2]

    def add_row(i, c):
      for j in range(DIM // LANES):
        s = pl.ds(j * LANES, LANES)
        plsc.addupdate(rows_v.at[i, s], pos_v[i, s])
      return c

    lax.fori_loop(0, CHUNK, add_row, 0)

  load_and_gather(0)
  for k in range(batch):
    if k + 1 < batch:
      if k >= 1:
        wait_out(k - 1)
      load_and_gather(k + 1)
    wait_gather(k)
    add_pos(k)
    write_out(k)
  wait_out(batch - 2)
  wait_out(batch - 1)


def kernel(idx, token_emb, pos_emb):
  B, T = idx.shape
  N = B * T
  info = plsc.get_sparse_core_info()
  num_workers = info.num_cores * info.num_subcores
  t_per_w = T // num_workers

  idx_flat = idx.reshape(N).astype(jnp.int32)

  mesh = plsc.VectorSubcoreMesh(core_axis_name="c", subcore_axis_name="s")
  run = functools.partial(
      pl.kernel,
      mesh=mesh,
      out_type=jax.ShapeDtypeStruct((N, DIM), jnp.float32),
      scratch_types=[
          pltpu.VMEM((CHUNK,), jnp.int32),
          pltpu.VMEM((CHUNK,), jnp.int32),
          pltpu.VMEM((CHUNK, DIM), jnp.float32),
          pltpu.VMEM((CHUNK, DIM), jnp.float32),
          pltpu.VMEM((CHUNK, DIM), jnp.float32),
          pltpu.SemaphoreType.DMA,
          pltpu.SemaphoreType.DMA,
      ],
  )(functools.partial(_emb_body, t_per_w, T, B, info.num_cores))

  out = run(idx_flat, token_emb, pos_emb)
  return out.reshape(B, T, DIM)


# speedup vs baseline: 1.5112x; 1.0907x over previous
import functools

import jax
import jax.numpy as jnp
from jax import lax
from jax.experimental import pallas as pl
from jax.experimental.pallas import tpu as pltpu
from jax.experimental.pallas import tpu_sc as plsc

DIM = 128
LANES = 16
CHUNK = 128
NBUF = 4


def _emb_body(t_per_w, seq_len, batch, num_cores,
              idx_hbm, tok_hbm, pos_hbm, out_hbm,
              idx_v, rows0, rows1, rows2, rows3, pos_v,
              sem_g, sem_o, sem_p):
  cid = lax.axis_index("c")
  sid = lax.axis_index("s")
  wid = sid * num_cores + cid
  t0 = wid * t_per_w
  n_rows = t_per_w * batch
  n_chunks = n_rows // CHUNK
  per_b = t_per_w // CHUNK

  row_bufs = (rows0, rows1, rows2, rows3)

  def hbm_off(k):
    b, h = k // per_b, k % per_b
    return b * seq_len + t0 + h * CHUNK

  def idx_slice(k):
    return idx_v.at[pl.ds(k * CHUNK, CHUNK)]

  def fire_gather(k):
    pltpu.async_copy(tok_hbm.at[idx_slice(k)], row_bufs[k % NBUF], sem_g)

  def wait_gather(k):
    pltpu.make_async_copy(
        tok_hbm.at[idx_slice(k)], row_bufs[k % NBUF], sem_g
    ).wait()

  def fire_write(k):
    pltpu.async_copy(row_bufs[k % NBUF], out_hbm.at[pl.ds(hbm_off(k), CHUNK)],
                     sem_o)

  def wait_write(k):
    pltpu.make_async_copy(
        row_bufs[k % NBUF], out_hbm.at[pl.ds(hbm_off(k), CHUNK)], sem_o
    ).wait()

  def add_pos(k):
    rows_v = row_bufs[k % NBUF]
    p0 = (k % per_b) * CHUNK

    def add_row(i, c):
      for j in range(DIM // LANES):
        s = pl.ds(j * LANES, LANES)
        plsc.addupdate(rows_v.at[i, s], pos_v[p0 + i, s])
      return c

    lax.fori_loop(0, CHUNK, add_row, 0)

  idx_cps = []
  for b in range(batch):
    cp = pltpu.make_async_copy(
        idx_hbm.at[pl.ds(b * seq_len + t0, t_per_w)],
        idx_v.at[pl.ds(b * t_per_w, t_per_w)],
        sem_p,
    )
    cp.start()
    idx_cps.append(cp)
  for cp in idx_cps:
    cp.wait()

  for k in range(NBUF - 1):
    fire_gather(k)
  pos_cp = pltpu.make_async_copy(pos_hbm.at[pl.ds(t0, t_per_w)], pos_v, sem_p)
  pos_cp.start()

  for k in range(n_chunks):
    wait_gather(k)
    if k == 0:
      pos_cp.wait()
    add_pos(k)
    fire_write(k)
    nk = k + NBUF - 1
    if nk < n_chunks:
      if k >= 1:
        wait_write(k - 1)
      fire_gather(nk)
  for k in range(n_chunks - NBUF, n_chunks):
    if k >= 0:
      wait_write(k)


def kernel(idx, token_emb, pos_emb):
  B, T = idx.shape
  N = B * T
  info = plsc.get_sparse_core_info()
  num_workers = info.num_cores * info.num_subcores
  t_per_w = T // num_workers

  idx_flat = idx.reshape(N).astype(jnp.int32)

  mesh = plsc.VectorSubcoreMesh(core_axis_name="c", subcore_axis_name="s")
  run = functools.partial(
      pl.kernel,
      mesh=mesh,
      out_type=jax.ShapeDtypeStruct((N, DIM), jnp.float32),
      scratch_types=[
          pltpu.VMEM((t_per_w * B,), jnp.int32),
          pltpu.VMEM((CHUNK, DIM), jnp.float32),
          pltpu.VMEM((CHUNK, DIM), jnp.float32),
          pltpu.VMEM((CHUNK, DIM), jnp.float32),
          pltpu.VMEM((CHUNK, DIM), jnp.float32),
          pltpu.VMEM((t_per_w, DIM), jnp.float32),
          pltpu.SemaphoreType.DMA,
          pltpu.SemaphoreType.DMA,
          pltpu.SemaphoreType.DMA,
      ],
  )(functools.partial(_emb_body, t_per_w, T, B, info.num_cores))

  out = run(idx_flat, token_emb, pos_emb)
  return out.reshape(B, T, DIM)


# speedup vs baseline: 1.5139x; 1.0018x over previous
import functools

import jax
import jax.numpy as jnp
from jax import lax
from jax.experimental import pallas as pl
from jax.experimental.pallas import tpu as pltpu
from jax.experimental.pallas import tpu_sc as plsc

DIM = 128
LANES = 16
CHUNK = 128
NBUF = 5


def _emb_body(t_per_w, seq_len, batch, num_cores,
              idx_hbm, tok_hbm, pos_hbm, out_hbm,
              idx_v, rows0, rows1, rows2, rows3, rows4, pos_v,
              sem_g, sem_o, sem_p):
  cid = lax.axis_index("c")
  sid = lax.axis_index("s")
  wid = sid * num_cores + cid
  t0 = wid * t_per_w
  n_rows = t_per_w * batch
  n_chunks = n_rows // CHUNK
  per_b = t_per_w // CHUNK

  row_bufs = (rows0, rows1, rows2, rows3, rows4)

  def hbm_off(k):
    b, h = k // per_b, k % per_b
    return b * seq_len + t0 + h * CHUNK

  def idx_slice(k):
    return idx_v.at[pl.ds(k * CHUNK, CHUNK)]

  def fire_gather(k):
    pltpu.async_copy(tok_hbm.at[idx_slice(k)], row_bufs[k % NBUF], sem_g)

  def wait_gather(k):
    pltpu.make_async_copy(
        tok_hbm.at[idx_slice(k)], row_bufs[k % NBUF], sem_g
    ).wait()

  def fire_write(k):
    pltpu.async_copy(row_bufs[k % NBUF], out_hbm.at[pl.ds(hbm_off(k), CHUNK)],
                     sem_o)

  def wait_write(k):
    pltpu.make_async_copy(
        row_bufs[k % NBUF], out_hbm.at[pl.ds(hbm_off(k), CHUNK)], sem_o
    ).wait()

  def add_pos(k):
    rows_v = row_bufs[k % NBUF]
    p0 = (k % per_b) * CHUNK

    def add_row(i, c):
      r = i * 2
      for u in range(2):
        for j in range(DIM // LANES):
          s = pl.ds(j * LANES, LANES)
          plsc.addupdate(rows_v.at[r + u, s], pos_v[p0 + r + u, s])
      return c

    lax.fori_loop(0, CHUNK // 2, add_row, 0)

  idx_cps = []
  for b in range(batch):
    cp = pltpu.make_async_copy(
        idx_hbm.at[pl.ds(b * seq_len + t0, t_per_w)],
        idx_v.at[pl.ds(b * t_per_w, t_per_w)],
        sem_p,
    )
    cp.start()
    idx_cps.append(cp)
  for cp in idx_cps:
    cp.wait()

  for k in range(NBUF - 1):
    fire_gather(k)
  pos_cp = pltpu.make_async_copy(pos_hbm.at[pl.ds(t0, t_per_w)], pos_v, sem_p)
  pos_cp.start()

  for k in range(n_chunks):
    wait_gather(k)
    if k == 0:
      pos_cp.wait()
    add_pos(k)
    fire_write(k)
    nk = k + NBUF - 1
    if nk < n_chunks:
      if k >= 1:
        wait_write(k - 1)
      fire_gather(nk)
  for k in range(n_chunks - NBUF, n_chunks):
    if k >= 0:
      wait_write(k)


def kernel(idx, token_emb, pos_emb):
  B, T = idx.shape
  N = B * T
  info = plsc.get_sparse_core_info()
  num_workers = info.num_cores * info.num_subcores
  t_per_w = T // num_workers

  idx_flat = idx.reshape(N).astype(jnp.int32)

  mesh = plsc.VectorSubcoreMesh(core_axis_name="c", subcore_axis_name="s")
  run = functools.partial(
      pl.kernel,
      mesh=mesh,
      out_type=jax.ShapeDtypeStruct((N, DIM), jnp.float32),
      scratch_types=[
          pltpu.VMEM((t_per_w * B,), jnp.int32),
          pltpu.VMEM((CHUNK, DIM), jnp.float32),
          pltpu.VMEM((CHUNK, DIM), jnp.float32),
          pltpu.VMEM((CHUNK, DIM), jnp.float32),
          pltpu.VMEM((CHUNK, DIM), jnp.float32),
          pltpu.VMEM((CHUNK, DIM), jnp.float32),
          pltpu.VMEM((t_per_w, DIM), jnp.float32),
          pltpu.SemaphoreType.DMA,
          pltpu.SemaphoreType.DMA,
          pltpu.SemaphoreType.DMA,
      ],
  )(functools.partial(_emb_body, t_per_w, T, B, info.num_cores))

  out = run(idx_flat, token_emb, pos_emb)
  return out.reshape(B, T, DIM)
